# baseline (speedup 1.0000x reference)
import functools

import jax
import jax.numpy as jnp
from jax import lax
from jax.experimental import pallas as pl
from jax.experimental.pallas import tpu as pltpu
from jax.experimental.pallas import tpu_sc as plsc

N = 10000
E = 320000
D = 128
A = 64

NC = 2
NS = 16
NW = NC * NS
E_PER_TILE = 10240
CHUNK = 80
STEPS = E_PER_TILE // CHUNK
E_PAD = NW * E_PER_TILE
NP = 10240
DB = NP // D
ROWS_PER_TILE = NP // NS


def _prep_body(x_ref, wo_ref, bo_ref, wa_ref, ba_ref, y_ref, p_ref):
    x = x_ref[...]
    h = lax.dot_general(x, wo_ref[...], (((1,), (1,)), ((), ())),
                        preferred_element_type=jnp.float32)
    h = h + bo_ref[...]
    s = jnp.sum(h * wa_ref[...], axis=1, keepdims=True) + ba_ref[0, 0]
    p = jnp.exp(s - jnp.max(s))
    y_ref[...] = x * p
    p_ref[...] = p


_prep = pl.pallas_call(
    _prep_body,
    out_shape=(jax.ShapeDtypeStruct((N, D), jnp.float32),
               jax.ShapeDtypeStruct((N, 1), jnp.float32)),
)


def _edge_body(y_hbm, p_hbm, src_hbm, dst_hbm, zeros_hbm, out_hbm,
               src_v, dst_v, rows_v, p_v, den_v, didx_v, accum_sh, sem):
    c = lax.axis_index("c")
    s = lax.axis_index("s")
    wid = s * NC + c

    pltpu.sync_copy(p_hbm, p_v)

    zero16 = jnp.zeros((16,), jnp.float32)
    lane = lax.iota(jnp.int32, 16)

    def zbody(i, carry):
        r = i // 8
        col = (i % 8) * 16
        den_v[r, pl.ds(col, 16)] = zero16
        return carry

    lax.fori_loop(0, DB * 8, zbody, 0)
    for g in range(DB // 16):
        didx_v[pl.ds(g * 16, 16)] = N + g * 16 + lane

    r0 = s * ROWS_PER_TILE
    pltpu.sync_copy(zeros_hbm.at[pl.ds(r0, ROWS_PER_TILE)],
                    accum_sh.at[pl.ds(r0, ROWS_PER_TILE)])
    plsc.subcore_barrier()

    base = pl.multiple_of(wid * E_PER_TILE, 8)

    def body(step, carry):
        off = pl.multiple_of(base + step * CHUNK, 8)
        pltpu.sync_copy(src_hbm.at[pl.ds(off, CHUNK)], src_v)
        pltpu.sync_copy(dst_hbm.at[pl.ds(off, CHUNK)], dst_v)
        cp = pltpu.async_copy(y_hbm.at[src_v], rows_v, sem)
        for i in range(CHUNK // 16):
            s16 = src_v[pl.ds(i * 16, 16)]
            d16 = dst_v[pl.ds(i * 16, 16)]
            p16 = plsc.load_gather(p_v, [s16])
            drow = lax.shift_right_logical(d16, 7)
            dcol = lax.bitwise_and(d16, 127)
            plsc.addupdate_scatter(den_v, [drow, dcol], p16)
        cp.wait()
        pltpu.sync_copy(rows_v, accum_sh.at[dst_v], add=True)
        return carry

    lax.fori_loop(0, STEPS, body, 0)

    pltpu.sync_copy(den_v, accum_sh.at[didx_v], add=True)
    plsc.subcore_barrier()

    o0 = pl.multiple_of(c * NP + r0, 8)
    pltpu.sync_copy(accum_sh.at[pl.ds(r0, ROWS_PER_TILE)],
                    out_hbm.at[pl.ds(o0, ROWS_PER_TILE)])


_edge_pass = functools.partial(
    pl.kernel,
    out_type=jax.ShapeDtypeStruct((NC * NP, D), jnp.float32),
    mesh=plsc.VectorSubcoreMesh(core_axis_name="c", subcore_axis_name="s"),
    compiler_params=pltpu.CompilerParams(needs_layout_passes=False),
    scratch_types=[
        pltpu.VMEM((CHUNK,), jnp.int32),
        pltpu.VMEM((CHUNK,), jnp.int32),
        pltpu.VMEM((CHUNK, D), jnp.float32),
        pltpu.VMEM((N,), jnp.float32),
        pltpu.VMEM((DB, D), jnp.float32),
        pltpu.VMEM((DB,), jnp.int32),
        pltpu.VMEM_SHARED((NP, D), jnp.float32),
        pltpu.SemaphoreType.DMA,
    ],
)(_edge_body)


def _finalize_body(pp_ref, o_ref):
    accf = pp_ref[0] + pp_ref[1]
    den = accf[N:N + DB]
    recip = 1.0 / jnp.where(den > 0.0, den, 1.0)
    eye = (lax.broadcasted_iota(jnp.int32, (D, D), 0)
           == lax.broadcasted_iota(jnp.int32, (D, D), 1)).astype(jnp.float32)
    for b in range(N // D + 1):
        nrows = min(D, N - b * D)
        diag = eye * recip[b:b + 1]
        blk = lax.dot_general(diag, accf[b * D:(b + 1) * D],
                              (((1,), (0,)), ((), ())),
                              preferred_element_type=jnp.float32)
        o_ref[pl.ds(b * D, nrows)] = blk[:nrows]


_finalize = pl.pallas_call(
    _finalize_body,
    out_shape=jax.ShapeDtypeStruct((N, D), jnp.float32),
)


def kernel(x, edge_index, W_o, b_o, W_a, b_a):
    src = edge_index[0]
    dst = edge_index[1]
    pad = E_PAD - E
    ar = jnp.arange(pad, dtype=jnp.int32)
    srcp = jnp.concatenate([src, ar % N])
    dstp = jnp.concatenate([dst, N + DB + ar % (NP - N - DB)])
    y, p = _prep(x, W_o, b_o.reshape(1, A), W_a, b_a.reshape(1, 1))
    zeros = jnp.zeros((NP, D), jnp.float32)
    partials = _edge_pass(y, p.reshape(N), srcp, dstp, zeros)
    return _finalize(partials.reshape(NC, NP, D))


# speedup vs baseline: 1.3892x; 1.3892x over previous
import functools

import jax
import jax.numpy as jnp
from jax import lax
from jax.experimental import pallas as pl
from jax.experimental.pallas import tpu as pltpu
from jax.experimental.pallas import tpu_sc as plsc

N = 10000
E = 320000
D = 128
A = 64

NC = 2
NS = 16
NW = NC * NS
E_PER_TILE = 10240
CHUNK = 80
STEPS = E_PER_TILE // CHUNK
E_PAD = NW * E_PER_TILE
NP = 10240
DB = NP // D
ROWS_PER_TILE = NP // NS


def _prep_body(x_ref, wo_ref, bo_ref, wa_ref, ba_ref, y_ref, p_ref):
    x = x_ref[...]
    h = lax.dot_general(x, wo_ref[...], (((1,), (1,)), ((), ())),
                        preferred_element_type=jnp.float32)
    h = h + bo_ref[...]
    s = jnp.sum(h * wa_ref[...], axis=1, keepdims=True) + ba_ref[0, 0]
    p = jnp.exp(s - jnp.max(s))
    y_ref[...] = x * p
    p_ref[...] = p


_prep = pl.pallas_call(
    _prep_body,
    out_shape=(jax.ShapeDtypeStruct((N, D), jnp.float32),
               jax.ShapeDtypeStruct((N, 1), jnp.float32)),
)


def _edge_body(y_hbm, p_hbm, src_hbm, dst_hbm, zeros_hbm, out_hbm,
               src_v, dst_v, src_w, dst_w, rows_v, p_v, den_v, didx_v, accum_sh,
               sem, semis0, semid0, semis1, semid1):
    c = lax.axis_index("c")
    s = lax.axis_index("s")
    wid = s * NC + c

    pltpu.sync_copy(p_hbm, p_v)

    zero16 = jnp.zeros((16,), jnp.float32)
    lane = lax.iota(jnp.int32, 16)

    def zbody(i, carry):
        r = i // 8
        col = (i % 8) * 16
        den_v[r, pl.ds(col, 16)] = zero16
        return carry

    lax.fori_loop(0, DB * 8, zbody, 0)
    for g in range(DB // 16):
        didx_v[pl.ds(g * 16, 16)] = N + g * 16 + lane

    r0 = s * ROWS_PER_TILE
    pltpu.sync_copy(zeros_hbm.at[pl.ds(r0, ROWS_PER_TILE)],
                    accum_sh.at[pl.ds(r0, ROWS_PER_TILE)])
    plsc.subcore_barrier()

    base = pl.multiple_of(wid * E_PER_TILE, 8)

    def eslice(hbm, g):
        return hbm.at[pl.ds(pl.multiple_of(base + g * CHUNK, 8), CHUNK)]

    def idx_fetch(g, sv, dv, semis, semid):
        pltpu.async_copy(eslice(src_hbm, g), sv, semis)
        pltpu.async_copy(eslice(dst_hbm, g), dv, semid)

    def idx_wait(g, sv, dv, semis, semid):
        pltpu.make_async_copy(eslice(src_hbm, g), sv, semis).wait()
        pltpu.make_async_copy(eslice(dst_hbm, g), dv, semid).wait()

    def den_update(sv, dv):
        for i in range(CHUNK // 16):
            s16 = sv[pl.ds(i * 16, 16)]
            d16 = dv[pl.ds(i * 16, 16)]
            p16 = plsc.load_gather(p_v, [s16])
            drow = lax.shift_right_logical(d16, 7)
            dcol = lax.bitwise_and(d16, 127)
            plsc.addupdate_scatter(den_v, [drow, dcol], p16)

    PAIRS = STEPS // 2
    idx_fetch(0, src_v, dst_v, semis0, semid0)
    idx_wait(0, src_v, dst_v, semis0, semid0)

    def body(h, carry):
        g0 = h * 2
        cp0 = pltpu.async_copy(y_hbm.at[src_v], rows_v, sem)
        idx_fetch(g0 + 1, src_w, dst_w, semis1, semid1)
        den_update(src_v, dst_v)
        cp0.wait()
        pltpu.sync_copy(rows_v, accum_sh.at[dst_v], add=True)
        idx_wait(g0 + 1, src_w, dst_w, semis1, semid1)
        cp1 = pltpu.async_copy(y_hbm.at[src_w], rows_v, sem)

        @pl.when(h + 1 < PAIRS)
        def _():
            idx_fetch(g0 + 2, src_v, dst_v, semis0, semid0)

        den_update(src_w, dst_w)
        cp1.wait()
        pltpu.sync_copy(rows_v, accum_sh.at[dst_w], add=True)

        @pl.when(h + 1 < PAIRS)
        def _():
            idx_wait(g0 + 2, src_v, dst_v, semis0, semid0)

        return carry

    lax.fori_loop(0, PAIRS, body, 0)

    pltpu.sync_copy(den_v, accum_sh.at[didx_v], add=True)
    plsc.subcore_barrier()

    o0 = pl.multiple_of(c * NP + r0, 8)
    pltpu.sync_copy(accum_sh.at[pl.ds(r0, ROWS_PER_TILE)],
                    out_hbm.at[pl.ds(o0, ROWS_PER_TILE)])


_edge_pass = functools.partial(
    pl.kernel,
    out_type=jax.ShapeDtypeStruct((NC * NP, D), jnp.float32),
    mesh=plsc.VectorSubcoreMesh(core_axis_name="c", subcore_axis_name="s"),
    compiler_params=pltpu.CompilerParams(needs_layout_passes=False),
    scratch_types=[
        pltpu.VMEM((CHUNK,), jnp.int32),
        pltpu.VMEM((CHUNK,), jnp.int32),
        pltpu.VMEM((CHUNK,), jnp.int32),
        pltpu.VMEM((CHUNK,), jnp.int32),
        pltpu.VMEM((CHUNK, D), jnp.float32),
        pltpu.VMEM((N,), jnp.float32),
        pltpu.VMEM((DB, D), jnp.float32),
        pltpu.VMEM((DB,), jnp.int32),
        pltpu.VMEM_SHARED((NP, D), jnp.float32),
        pltpu.SemaphoreType.DMA,
        pltpu.SemaphoreType.DMA,
        pltpu.SemaphoreType.DMA,
        pltpu.SemaphoreType.DMA,
        pltpu.SemaphoreType.DMA,
    ],
)(_edge_body)


def _finalize_body(pp_ref, o_ref):
    accf = pp_ref[0] + pp_ref[1]
    den = accf[N:N + DB]
    recip = 1.0 / jnp.where(den > 0.0, den, 1.0)
    eye = (lax.broadcasted_iota(jnp.int32, (D, D), 0)
           == lax.broadcasted_iota(jnp.int32, (D, D), 1)).astype(jnp.float32)
    for b in range(N // D + 1):
        nrows = min(D, N - b * D)
        diag = eye * recip[b:b + 1]
        blk = lax.dot_general(diag, accf[b * D:(b + 1) * D],
                              (((1,), (0,)), ((), ())),
                              preferred_element_type=jnp.float32)
        o_ref[pl.ds(b * D, nrows)] = blk[:nrows]


_finalize = pl.pallas_call(
    _finalize_body,
    out_shape=jax.ShapeDtypeStruct((N, D), jnp.float32),
)


def kernel(x, edge_index, W_o, b_o, W_a, b_a):
    src = edge_index[0]
    dst = edge_index[1]
    pad = E_PAD - E
    ar = jnp.arange(pad, dtype=jnp.int32)
    srcp = jnp.concatenate([src, ar % N])
    dstp = jnp.concatenate([dst, N + DB + ar % (NP - N - DB)])
    y, p = _prep(x, W_o, b_o.reshape(1, A), W_a, b_a.reshape(1, 1))
    zeros = jnp.zeros((NP, D), jnp.float32)
    partials = _edge_pass(y, p.reshape(N), srcp, dstp, zeros)
    return _finalize(partials.reshape(NC, NP, D))


# speedup vs baseline: 1.7794x; 1.2808x over previous
import functools

import jax
import jax.numpy as jnp
from jax import lax
from jax.experimental import pallas as pl
from jax.experimental.pallas import tpu as pltpu
from jax.experimental.pallas import tpu_sc as plsc

N = 10000
E = 320000
D = 128
A = 64

NC = 2
NS = 16
NW = NC * NS
E_PER_TILE = 10240
CHUNK = 80
STEPS = E_PER_TILE // CHUNK
E_PAD = NW * E_PER_TILE
NP = 10240
DB = NP // D
ROWS_PER_TILE = NP // NS


def _prep_body(x_ref, wo_ref, bo_ref, wa_ref, ba_ref, y_ref, p_ref):
    x = x_ref[...]
    h = lax.dot_general(x, wo_ref[...], (((1,), (1,)), ((), ())),
                        preferred_element_type=jnp.float32)
    h = h + bo_ref[...]
    s = jnp.sum(h * wa_ref[...], axis=1, keepdims=True) + ba_ref[0, 0]
    p = jnp.exp(s - jnp.max(s))
    y_ref[...] = x * p
    p_ref[...] = p


_prep = pl.pallas_call(
    _prep_body,
    out_shape=(jax.ShapeDtypeStruct((N, D), jnp.float32),
               jax.ShapeDtypeStruct((N, 1), jnp.float32)),
)


def _edge_body(y_hbm, p_hbm, src_hbm, dst_hbm, zeros_hbm, out_hbm,
               src_v, dst_v, src_w, dst_w, rows_v, rows_w, p_v, den_v, didx_v,
               accum_sh, sem, semr1, semis0, semid0, semis1, semid1):
    c = lax.axis_index("c")
    s = lax.axis_index("s")
    wid = s * NC + c

    pltpu.sync_copy(p_hbm, p_v)

    zero16 = jnp.zeros((16,), jnp.float32)
    lane = lax.iota(jnp.int32, 16)

    def zbody(i, carry):
        r = i // 8
        col = (i % 8) * 16
        den_v[r, pl.ds(col, 16)] = zero16
        return carry

    lax.fori_loop(0, DB * 8, zbody, 0)
    for g in range(DB // 16):
        didx_v[pl.ds(g * 16, 16)] = N + g * 16 + lane

    r0 = s * ROWS_PER_TILE
    pltpu.sync_copy(zeros_hbm.at[pl.ds(r0, ROWS_PER_TILE)],
                    accum_sh.at[pl.ds(r0, ROWS_PER_TILE)])
    plsc.subcore_barrier()

    base = pl.multiple_of(wid * E_PER_TILE, 8)

    def eslice(hbm, g):
        return hbm.at[pl.ds(pl.multiple_of(base + g * CHUNK, 8), CHUNK)]

    def idx_fetch(g, sv, dv, semis, semid):
        pltpu.async_copy(eslice(src_hbm, g), sv, semis)
        pltpu.async_copy(eslice(dst_hbm, g), dv, semid)

    def idx_wait(g, sv, dv, semis, semid):
        pltpu.make_async_copy(eslice(src_hbm, g), sv, semis).wait()
        pltpu.make_async_copy(eslice(dst_hbm, g), dv, semid).wait()

    def den_update(sv, dv):
        for i in range(CHUNK // 16):
            s16 = sv[pl.ds(i * 16, 16)]
            d16 = dv[pl.ds(i * 16, 16)]
            p16 = plsc.load_gather(p_v, [s16])
            drow = lax.shift_right_logical(d16, 7)
            dcol = lax.bitwise_and(d16, 127)
            plsc.addupdate_scatter(den_v, [drow, dcol], p16)

    PAIRS = STEPS // 2
    idx_fetch(0, src_v, dst_v, semis0, semid0)
    idx_wait(0, src_v, dst_v, semis0, semid0)
    pltpu.async_copy(y_hbm.at[src_v], rows_v, sem)
    idx_fetch(1, src_w, dst_w, semis1, semid1)

    def body(h, carry):
        g0 = h * 2
        idx_wait(g0 + 1, src_w, dst_w, semis1, semid1)
        pltpu.async_copy(y_hbm.at[src_w], rows_w, semr1)
        den_update(src_v, dst_v)
        pltpu.make_async_copy(y_hbm.at[src_v], rows_v, sem).wait()
        pltpu.sync_copy(rows_v, accum_sh.at[dst_v], add=True)

        @pl.when(h + 1 < PAIRS)
        def _():
            idx_fetch(g0 + 2, src_v, dst_v, semis0, semid0)

        den_update(src_w, dst_w)
        pltpu.make_async_copy(y_hbm.at[src_w], rows_w, semr1).wait()
        pltpu.sync_copy(rows_w, accum_sh.at[dst_w], add=True)

        @pl.when(h + 1 < PAIRS)
        def _():
            idx_wait(g0 + 2, src_v, dst_v, semis0, semid0)
            pltpu.async_copy(y_hbm.at[src_v], rows_v, sem)
            idx_fetch(g0 + 3, src_w, dst_w, semis1, semid1)

        return carry

    lax.fori_loop(0, PAIRS, body, 0)

    pltpu.sync_copy(den_v, accum_sh.at[didx_v], add=True)
    plsc.subcore_barrier()

    o0 = pl.multiple_of(c * NP + r0, 8)
    pltpu.sync_copy(accum_sh.at[pl.ds(r0, ROWS_PER_TILE)],
                    out_hbm.at[pl.ds(o0, ROWS_PER_TILE)])


_edge_pass = functools.partial(
    pl.kernel,
    out_type=jax.ShapeDtypeStruct((NC * NP, D), jnp.float32),
    mesh=plsc.VectorSubcoreMesh(core_axis_name="c", subcore_axis_name="s"),
    compiler_params=pltpu.CompilerParams(needs_layout_passes=False),
    scratch_types=[
        pltpu.VMEM((CHUNK,), jnp.int32),
        pltpu.VMEM((CHUNK,), jnp.int32),
        pltpu.VMEM((CHUNK,), jnp.int32),
        pltpu.VMEM((CHUNK,), jnp.int32),
        pltpu.VMEM((CHUNK, D), jnp.float32),
        pltpu.VMEM((CHUNK, D), jnp.float32),
        pltpu.VMEM((N,), jnp.float32),
        pltpu.VMEM((DB, D), jnp.float32),
        pltpu.VMEM((DB,), jnp.int32),
        pltpu.VMEM_SHARED((NP, D), jnp.float32),
        pltpu.SemaphoreType.DMA,
        pltpu.SemaphoreType.DMA,
        pltpu.SemaphoreType.DMA,
        pltpu.SemaphoreType.DMA,
        pltpu.SemaphoreType.DMA,
        pltpu.SemaphoreType.DMA,
    ],
)(_edge_body)


def _finalize_body(pp_ref, o_ref):
    accf = pp_ref[0] + pp_ref[1]
    den = accf[N:N + DB]
    recip = 1.0 / jnp.where(den > 0.0, den, 1.0)
    eye = (lax.broadcasted_iota(jnp.int32, (D, D), 0)
           == lax.broadcasted_iota(jnp.int32, (D, D), 1)).astype(jnp.float32)
    for b in range(N // D + 1):
        nrows = min(D, N - b * D)
        diag = eye * recip[b:b + 1]
        blk = lax.dot_general(diag, accf[b * D:(b + 1) * D],
                              (((1,), (0,)), ((), ())),
                              preferred_element_type=jnp.float32)
        o_ref[pl.ds(b * D, nrows)] = blk[:nrows]


_finalize = pl.pallas_call(
    _finalize_body,
    out_shape=jax.ShapeDtypeStruct((N, D), jnp.float32),
)


def kernel(x, edge_index, W_o, b_o, W_a, b_a):
    src = edge_index[0]
    dst = edge_index[1]
    pad = E_PAD - E
    ar = jnp.arange(pad, dtype=jnp.int32)
    srcp = jnp.concatenate([src, ar % N])
    dstp = jnp.concatenate([dst, N + DB + ar % (NP - N - DB)])
    y, p = _prep(x, W_o, b_o.reshape(1, A), W_a, b_a.reshape(1, 1))
    zeros = jnp.zeros((NP, D), jnp.float32)
    partials = _edge_pass(y, p.reshape(N), srcp, dstp, zeros)
    return _finalize(partials.reshape(NC, NP, D))
